# baseline (device time: 22772 ns/iter reference)
import jax
import jax.numpy as jnp
from jax import lax
from jax.experimental import pallas as pl
from jax.experimental.pallas import tpu as pltpu

N_DEV = 4
B = 2
SQ_LOC = 128
SKV = 128
H_LOC = 4
DH = 64
D_MODEL = 512
ROWS = B * SQ_LOC
HD_LOC = H_LOC * DH
KCAT = H_LOC * SKV
NEG = -1e9


def _body(x_ref, wq_ref, kbig_ref, vbig_ref, wo_ref, out_ref,
          xcomm, pcomm, psend, ctx_ref,
          x_ssem, x_rsem, p_ssem, p_rsem):
    my = lax.axis_index("i")
    even = (my % 2) == 0

    bar = pltpu.get_barrier_semaphore()
    for d in range(1, N_DEV):
        pl.semaphore_signal(
            bar, inc=1,
            device_id=((my + d) % N_DEV,),
            device_id_type=pl.DeviceIdType.MESH,
        )
    pl.semaphore_wait(bar, N_DEV - 1)

    @pl.when(even)
    def _():
        for d in range(1, N_DEV):
            for b in range(B):
                rdma = pltpu.make_async_remote_copy(
                    src_ref=x_ref.at[b],
                    dst_ref=xcomm.at[my // 2, b],
                    send_sem=x_ssem.at[d - 1, b],
                    recv_sem=x_rsem.at[my // 2, b],
                    device_id=((my + d) % N_DEV,),
                    device_id_type=pl.DeviceIdType.MESH,
                )
                rdma.start()

    rq = lax.broadcasted_iota(jnp.int32, (SQ_LOC, KCAT), 0)
    cc = lax.broadcasted_iota(jnp.int32, (SQ_LOC, KCAT), 1)
    maskb = (rq // 64) == ((cc % SKV) // 64)
    er = lax.broadcasted_iota(jnp.int32, (KCAT, H_LOC), 0)
    ec = lax.broadcasted_iota(jnp.int32, (KCAT, H_LOC), 1)
    eones = jnp.where((er // SKV) == ec, 1.0, 0.0).astype(jnp.bfloat16)
    tr = lax.broadcasted_iota(jnp.int32, (H_LOC, HD_LOC), 0)
    tc = lax.broadcasted_iota(jnp.int32, (H_LOC, HD_LOC), 1)
    et = jnp.where(tr == (tc // DH), 1.0, 0.0)

    def half_block(xslab, b):
        q2 = (jnp.dot(xslab, wq_ref[...], preferred_element_type=jnp.float32)
              * 0.125).astype(jnp.bfloat16)
        s = jnp.dot(q2, kbig_ref[b],
                    preferred_element_type=jnp.float32)
        w = jnp.exp(jnp.where(maskb, s, NEG)).astype(jnp.bfloat16)
        ws = jnp.dot(w, eones, preferred_element_type=jnp.float32)
        ctx_raw = jnp.dot(w, vbig_ref[b],
                          preferred_element_type=jnp.float32)
        rexp = jnp.dot(1.0 / ws, et,
                       preferred_element_type=jnp.float32)
        ctx_ref[...] = (ctx_raw * rexp).astype(jnp.bfloat16)
        return jnp.dot(ctx_ref[...], wo_ref[...],
                       preferred_element_type=jnp.float32)

    for g in (0, 2):
        @pl.when(my == g)
        def _():
            for b in range(B):
                out_ref[b * SQ_LOC:(b + 1) * SQ_LOC, :] = \
                    half_block(x_ref[b], b)

    @pl.when(jnp.logical_not(even))
    def _():
        out_ref[...] = jnp.zeros((ROWS, D_MODEL), jnp.float32)

    def remote_partial(g):
        for b in range(B):
            xr = pltpu.make_async_remote_copy(
                src_ref=x_ref.at[b],
                dst_ref=xcomm.at[g // 2, b],
                send_sem=x_ssem.at[0, b],
                recv_sem=x_rsem.at[g // 2, b],
                device_id=(g,),
                device_id_type=pl.DeviceIdType.MESH,
            )
            xr.wait_recv()
            psend[g // 2, b] = half_block(xcomm[g // 2, b], b).astype(jnp.bfloat16)
            send = pltpu.make_async_remote_copy(
                src_ref=psend.at[g // 2, b],
                dst_ref=pcomm.at[my, b],
                send_sem=p_ssem.at[g // 2, b],
                recv_sem=p_rsem.at[my, b],
                device_id=(g,),
                device_id_type=pl.DeviceIdType.MESH,
            )
            send.start()

    @pl.when(my == 0)
    def _():
        remote_partial(2)

    @pl.when(my == 2)
    def _():
        remote_partial(0)

    @pl.when(my == 1)
    def _():
        remote_partial(0)
        remote_partial(2)

    @pl.when(my == 3)
    def _():
        remote_partial(2)
        remote_partial(0)

    for s in range(N_DEV):
        @pl.when(even & (my != s))
        def _():
            for b in range(B):
                pr = pltpu.make_async_remote_copy(
                    src_ref=psend.at[0, b],
                    dst_ref=pcomm.at[s, b],
                    send_sem=p_ssem.at[0, b],
                    recv_sem=p_rsem.at[s, b],
                    device_id=(s,),
                    device_id_type=pl.DeviceIdType.MESH,
                )
                pr.wait_recv()
                out_ref[b * SQ_LOC:(b + 1) * SQ_LOC, :] = (
                    out_ref[b * SQ_LOC:(b + 1) * SQ_LOC, :]
                    + pcomm[s, b].astype(jnp.float32)
                )

    @pl.when(even)
    def _():
        for d in range(1, N_DEV):
            for b in range(B):
                pltpu.make_async_remote_copy(
                    src_ref=x_ref.at[b], dst_ref=xcomm.at[0, b],
                    send_sem=x_ssem.at[d - 1, b], recv_sem=x_rsem.at[0, b],
                    device_id=((my + d) % N_DEV,),
                    device_id_type=pl.DeviceIdType.MESH,
                ).wait_send()

    for g in (0, 2):
        @pl.when(my != g)
        def _():
            for b in range(B):
                pltpu.make_async_remote_copy(
                    src_ref=psend.at[g // 2, b], dst_ref=pcomm.at[my, b],
                    send_sem=p_ssem.at[g // 2, b], recv_sem=p_rsem.at[my, b],
                    device_id=(g,),
                    device_id_type=pl.DeviceIdType.MESH,
                ).wait_send()


def kernel(x, Wq, K_ext, V_ext, Wo):
    my = lax.axis_index("i")
    xb = x.astype(jnp.bfloat16)
    wqb = Wq.astype(jnp.bfloat16)
    wob = Wo.astype(jnp.bfloat16)
    k_loc = lax.dynamic_slice_in_dim(K_ext, my * H_LOC, H_LOC, axis=2)
    v_loc = lax.dynamic_slice_in_dim(V_ext, my * H_LOC, H_LOC, axis=2)
    kt = jnp.transpose(k_loc, (0, 2, 3, 1))
    vt = jnp.transpose(v_loc, (0, 2, 1, 3))
    kbig = jnp.zeros((B, H_LOC, DH, H_LOC, SKV), jnp.float32)
    vbig = jnp.zeros((B, H_LOC, SKV, H_LOC, DH), jnp.float32)
    for hh in range(H_LOC):
        kbig = kbig.at[:, hh, :, hh, :].set(kt[:, hh])
        vbig = vbig.at[:, hh, :, hh, :].set(vt[:, hh])
    kbig = kbig.reshape(B, HD_LOC, KCAT).astype(jnp.bfloat16)
    vbig = vbig.reshape(B, KCAT, HD_LOC).astype(jnp.bfloat16)

    out2 = pl.pallas_call(
        _body,
        out_shape=jax.ShapeDtypeStruct((ROWS, D_MODEL), jnp.float32),
        in_specs=[pl.BlockSpec(memory_space=pltpu.VMEM)] * 5,
        out_specs=pl.BlockSpec(memory_space=pltpu.VMEM),
        scratch_shapes=[
            pltpu.VMEM((2, B, SQ_LOC, D_MODEL), jnp.bfloat16),
            pltpu.VMEM((N_DEV, B, SQ_LOC, D_MODEL), jnp.bfloat16),
            pltpu.VMEM((2, B, SQ_LOC, D_MODEL), jnp.bfloat16),
            pltpu.VMEM((SQ_LOC, HD_LOC), jnp.bfloat16),
            pltpu.SemaphoreType.DMA((N_DEV - 1, B)),
            pltpu.SemaphoreType.DMA((2, B)),
            pltpu.SemaphoreType.DMA((2, B)),
            pltpu.SemaphoreType.DMA((N_DEV, B)),
        ],
        compiler_params=pltpu.CompilerParams(collective_id=0),
    )(xb, wqb, kbig, vbig, wob)
    return out2.reshape(B, SQ_LOC, D_MODEL)
